# Initial kernel scaffold; baseline (speedup 1.0000x reference)
import jax
import jax.numpy as jnp
from jax.experimental import pallas as pl


def kernel(x, table):
    raise NotImplementedError("write your pallas kernel here")


# speedup vs baseline: 1.6833x; 1.6833x over previous
import functools

import jax
import jax.numpy as jnp
from jax import lax
from jax.experimental import pallas as pl
from jax.experimental.pallas import tpu as pltpu
from jax.experimental.pallas import tpu_sc as plsc


def _emb_body(n_ch, ch, d, idx_hbm, table_hbm, out_hbm, idx_v, rows_v, gsem):
    nc = 2
    wid = lax.axis_index("s") * nc + lax.axis_index("c")
    b_per_w = n_ch * ch
    base = wid * b_per_w
    pltpu.sync_copy(idx_hbm.at[wid], idx_v)

    def chunk(j, carry):
        pltpu.async_copy(table_hbm.at[idx_v.at[j]], rows_v, gsem).wait()
        pltpu.sync_copy(rows_v, out_hbm.at[pl.ds(base + j * ch, ch)])
        return carry

    lax.fori_loop(0, n_ch, chunk, 0)


def kernel(x, table):
    b0, b1 = x.shape
    v, d = table.shape
    b = b0 * b1
    nw = 32
    ch = 128
    b_per_w = b // nw
    n_ch = b_per_w // ch
    assert b_per_w * nw == b and n_ch * ch == b_per_w

    idx = x.reshape(nw, n_ch, ch).astype(jnp.int32)

    mesh = plsc.VectorSubcoreMesh(core_axis_name="c", subcore_axis_name="s")
    emb = functools.partial(
        pl.kernel,
        mesh=mesh,
        out_type=jax.ShapeDtypeStruct((b, d), jnp.float32),
        scratch_types=[
            pltpu.VMEM((n_ch, ch), jnp.int32),
            pltpu.VMEM((ch, d), jnp.float32),
            pltpu.SemaphoreType.DMA,
        ],
        compiler_params=pltpu.CompilerParams(use_tc_tiling_on_sc=False),
    )(functools.partial(_emb_body, n_ch, ch, d))

    out = emb(idx, table)
    return out.reshape(b0, b1, d)


# speedup vs baseline: 1.8772x; 1.1152x over previous
import functools

import jax
import jax.numpy as jnp
from jax import lax
from jax.experimental import pallas as pl
from jax.experimental.pallas import tpu as pltpu
from jax.experimental.pallas import tpu_sc as plsc

NBUF = 4


def _emb_body(n_ch, ch, d, idx_hbm, table_hbm, out_hbm, idx_v, rows_v, *sems):
    gsem = sems[:NBUF]
    wsem = sems[NBUF:]
    nc = 2
    wid = lax.axis_index("s") * nc + lax.axis_index("c")
    b_per_w = n_ch * ch
    base = wid * b_per_w
    pltpu.sync_copy(idx_hbm.at[wid], idx_v)

    def gather(j, b):
        return pltpu.make_async_copy(table_hbm.at[idx_v.at[j]], rows_v.at[b],
                                     gsem[b])

    def write(j, b):
        return pltpu.make_async_copy(rows_v.at[b],
                                     out_hbm.at[pl.ds(base + j * ch, ch)],
                                     wsem[b])

    for b in range(NBUF):
        gather(b, b).start()

    n_rounds = n_ch // NBUF

    def steady(t, carry):
        j0 = t * NBUF
        for b in range(NBUF):
            j = j0 + b
            gather(j, b).wait()
            write(j, b).start()
            write(j, b).wait()
            gather(j + NBUF, b).start()
        return carry

    lax.fori_loop(0, n_rounds - 1, steady, 0)

    j0 = (n_rounds - 1) * NBUF
    for b in range(NBUF):
        j = j0 + b
        gather(j, b).wait()
        write(j, b).start()
        write(j, b).wait()


def kernel(x, table):
    b0, b1 = x.shape
    v, d = table.shape
    b = b0 * b1
    nw = 32
    ch = 128
    b_per_w = b // nw
    n_ch = b_per_w // ch
    assert b_per_w * nw == b and n_ch * ch == b_per_w and n_ch % NBUF == 0

    idx = x.reshape(nw, n_ch, ch).astype(jnp.int32)

    mesh = plsc.VectorSubcoreMesh(core_axis_name="c", subcore_axis_name="s")
    emb = functools.partial(
        pl.kernel,
        mesh=mesh,
        out_type=jax.ShapeDtypeStruct((b, d), jnp.float32),
        scratch_types=(
            [pltpu.VMEM((n_ch, ch), jnp.int32),
             pltpu.VMEM((NBUF, ch, d), jnp.float32)]
            + [pltpu.SemaphoreType.DMA] * (2 * NBUF)
        ),
        compiler_params=pltpu.CompilerParams(use_tc_tiling_on_sc=False),
    )(functools.partial(_emb_body, n_ch, ch, d))

    out = emb(idx, table)
    return out.reshape(b0, b1, d)
